# baseline (speedup 1.0000x reference)
import functools

import jax
import jax.numpy as jnp
from jax.experimental import pallas as pl
from jax.experimental.pallas import tpu as pltpu

_D, _H, _E = 768, 512, 8
_TB = 512


def _moe_block(x_ref, g_ref, b_ref, wg_ref, bg_ref, w1_hbm, b1_ref, w2_hbm,
               b2_ref, out_ref, gw_ref, w1_v, w2_v, sems):
    i = pl.program_id(0)

    @pl.when(i == 0)
    def _start_weight_copies():
        for e in range(_E):
            pltpu.make_async_copy(w1_hbm.at[e], w1_v.at[e],
                                  sems.at[0, e]).start()
            pltpu.make_async_copy(w2_hbm.at[e], w2_v.at[e],
                                  sems.at[1, e]).start()

    xb = x_ref[...]
    mu = jnp.mean(xb, axis=-1, keepdims=True)
    xc = xb - mu
    var = jnp.mean(xc * xc, axis=-1, keepdims=True)
    ln = xc / jnp.sqrt(var + 1e-5) * g_ref[...] + b_ref[...]
    logits = jnp.dot(ln, wg_ref[...], preferred_element_type=jnp.float32)
    logits = logits + bg_ref[...]
    logits = logits - jnp.max(logits, axis=-1, keepdims=True)
    ex = jnp.exp(logits)
    gw = ex / jnp.sum(ex, axis=-1, keepdims=True)
    gw_ref[...] = gw

    idx = jax.lax.broadcasted_iota(jnp.int32, gw.shape, 1)
    m1 = jnp.max(gw, axis=-1, keepdims=True)
    i1 = jnp.min(jnp.where(gw == m1, idx, _E), axis=-1, keepdims=True)
    sel1 = idx == i1
    gw2 = jnp.where(sel1, -1.0, gw)
    m2 = jnp.max(gw2, axis=-1, keepdims=True)
    i2 = jnp.min(jnp.where(gw2 == m2, idx, _E), axis=-1, keepdims=True)
    sel2 = idx == i2
    scale = (jnp.where(sel1, m1, 0.0) + jnp.where(sel2, m2, 0.0)) / (m1 + m2)

    acc = jnp.zeros((xb.shape[0], _D), jnp.float32)
    for e in range(_E):
        @pl.when(i == 0)
        def _wait_w1():
            pltpu.make_async_copy(w1_hbm.at[e], w1_v.at[e],
                                  sems.at[0, e]).wait()

        h = jnp.dot(xb, w1_v[e], preferred_element_type=jnp.float32)
        h = h + b1_ref[e]
        h = 0.5 * h * (1.0 + jax.lax.erf(h * 0.7071067811865476))

        @pl.when(i == 0)
        def _wait_w2():
            pltpu.make_async_copy(w2_hbm.at[e], w2_v.at[e],
                                  sems.at[1, e]).wait()

        o = jnp.dot(h, w2_v[e], preferred_element_type=jnp.float32)
        o = o + b2_ref[e]
        acc = acc + scale[:, e:e + 1] * o
    out_ref[...] = acc


@functools.partial(jax.jit, static_argnames=("interpret",))
def _moe(x_flat, g2, b2d, wg, bg2, w1, b1, w2, b2, interpret=False):
    t = x_flat.shape[0]
    grid = (t // _TB,)
    out, gw = pl.pallas_call(
        _moe_block,
        grid=grid,
        in_specs=[
            pl.BlockSpec((_TB, _D), lambda i: (i, 0)),
            pl.BlockSpec((1, _D), lambda i: (0, 0)),
            pl.BlockSpec((1, _D), lambda i: (0, 0)),
            pl.BlockSpec((_D, _E), lambda i: (0, 0)),
            pl.BlockSpec((1, _E), lambda i: (0, 0)),
            pl.BlockSpec(memory_space=pltpu.MemorySpace.HBM),
            pl.BlockSpec((_E, _H), lambda i: (0, 0)),
            pl.BlockSpec(memory_space=pltpu.MemorySpace.HBM),
            pl.BlockSpec((_E, _D), lambda i: (0, 0)),
        ],
        out_specs=[
            pl.BlockSpec((_TB, _D), lambda i: (i, 0)),
            pl.BlockSpec((_TB, _E), lambda i: (i, 0)),
        ],
        out_shape=[
            jax.ShapeDtypeStruct((t, _D), jnp.float32),
            jax.ShapeDtypeStruct((t, _E), jnp.float32),
        ],
        scratch_shapes=[
            pltpu.VMEM((_E, _D, _H), jnp.float32),
            pltpu.VMEM((_E, _H, _D), jnp.float32),
            pltpu.SemaphoreType.DMA((2, _E)),
        ],
        compiler_params=pltpu.CompilerParams(
            dimension_semantics=("arbitrary",),
            vmem_limit_bytes=100 * 1024 * 1024,
        ),
        interpret=interpret,
    )(x_flat, g2, b2d, wg, bg2, w1, b1, w2, b2)
    return out, gw


def kernel(x, ln_gamma, ln_beta, Wg, bg, W1, b1, W2, b2):
    b, l, d = x.shape
    x_flat = x.reshape(-1, d)
    out, gw = _moe(x_flat, ln_gamma.reshape(1, -1), ln_beta.reshape(1, -1),
                   Wg, bg.reshape(1, -1), W1, b1, W2, b2)
    return out.reshape(b, l, d), gw


# speedup vs baseline: 1.4651x; 1.4651x over previous
import functools

import jax
import jax.numpy as jnp
from jax.experimental import pallas as pl
from jax.experimental.pallas import tpu as pltpu

_D, _H, _E = 768, 512, 8
_TB = 512


def _moe_block(x_ref, g_ref, b_ref, wg_ref, bg_ref, w1_ref, b1_ref, w2_ref,
               b2_ref, out_ref, gw_ref):
    xb = x_ref[...]
    mu = jnp.mean(xb, axis=-1, keepdims=True)
    xc = xb - mu
    var = jnp.mean(xc * xc, axis=-1, keepdims=True)
    ln = xc * (jax.lax.rsqrt(var + 1e-5) * g_ref[...]) + b_ref[...]
    logits = jnp.dot(ln, wg_ref[...], preferred_element_type=jnp.float32)
    logits = logits + bg_ref[...]
    logits = logits - jnp.max(logits, axis=-1, keepdims=True)
    ex = jnp.exp(logits)
    gw = ex / jnp.sum(ex, axis=-1, keepdims=True)
    gw_ref[...] = gw

    idx = jax.lax.broadcasted_iota(jnp.int32, gw.shape, 1)
    m1 = jnp.max(gw, axis=-1, keepdims=True)
    i1 = jnp.min(jnp.where(gw == m1, idx, _E), axis=-1, keepdims=True)
    sel1 = idx == i1
    gw2 = jnp.where(sel1, -1.0, gw)
    m2 = jnp.max(gw2, axis=-1, keepdims=True)
    i2 = jnp.min(jnp.where(gw2 == m2, idx, _E), axis=-1, keepdims=True)
    sel2 = idx == i2
    scale = (jnp.where(sel1, m1, 0.0) + jnp.where(sel2, m2, 0.0)) / (m1 + m2)

    acc = jnp.zeros((xb.shape[0], _D), jnp.float32)
    for e in range(_E):
        h = jnp.dot(xb, w1_ref[e], preferred_element_type=jnp.float32)
        h = h + b1_ref[e]
        h = 0.5 * h * (1.0 + jax.lax.erf(h * 0.7071067811865476))
        o = jnp.dot(h, w2_ref[e], preferred_element_type=jnp.float32)
        o = o + b2_ref[e]
        acc = acc + scale[:, e:e + 1] * o
    out_ref[...] = acc


@functools.partial(jax.jit, static_argnames=("interpret",))
def _moe(x_flat, g2, b2d, wg, bg2, w1, b1, w2, b2, interpret=False):
    t = x_flat.shape[0]
    grid = (t // _TB,)
    out, gw = pl.pallas_call(
        _moe_block,
        grid=grid,
        in_specs=[
            pl.BlockSpec((_TB, _D), lambda i: (i, 0)),
            pl.BlockSpec((1, _D), lambda i: (0, 0)),
            pl.BlockSpec((1, _D), lambda i: (0, 0)),
            pl.BlockSpec((_D, _E), lambda i: (0, 0)),
            pl.BlockSpec((1, _E), lambda i: (0, 0)),
            pl.BlockSpec((_E, _D, _H), lambda i: (0, 0, 0)),
            pl.BlockSpec((_E, _H), lambda i: (0, 0)),
            pl.BlockSpec((_E, _H, _D), lambda i: (0, 0, 0)),
            pl.BlockSpec((_E, _D), lambda i: (0, 0)),
        ],
        out_specs=[
            pl.BlockSpec((_TB, _D), lambda i: (i, 0)),
            pl.BlockSpec((_TB, _E), lambda i: (i, 0)),
        ],
        out_shape=[
            jax.ShapeDtypeStruct((t, _D), jnp.float32),
            jax.ShapeDtypeStruct((t, _E), jnp.float32),
        ],
        compiler_params=pltpu.CompilerParams(
            dimension_semantics=("parallel",),
            vmem_limit_bytes=100 * 1024 * 1024,
        ),
        interpret=interpret,
    )(x_flat, g2, b2d, wg, bg2, w1, b1, w2, b2)
    return out, gw


def kernel(x, ln_gamma, ln_beta, Wg, bg, W1, b1, W2, b2):
    b, l, d = x.shape
    x_flat = x.reshape(-1, d)
    out, gw = _moe(x_flat, ln_gamma.reshape(1, -1), ln_beta.reshape(1, -1),
                   Wg, bg.reshape(1, -1), W1, b1, W2, b2)
    return out.reshape(b, l, d), gw
